# Initial kernel scaffold; baseline (speedup 1.0000x reference)
import jax
import jax.numpy as jnp
from jax.experimental import pallas as pl


def kernel(input, embd_weight):
    raise NotImplementedError("write your pallas kernel here")


# speedup vs baseline: 5.8015x; 5.8015x over previous
import functools

import jax
import jax.numpy as jnp
from jax import lax
from jax.experimental import pallas as pl
from jax.experimental.pallas import tpu as pltpu
from jax.experimental.pallas import tpu_sc as plsc

_VOCAB = 100000
_DIM = 64
_B = 16384
_L = 50
_N = _B * _L
_NW = 32
_PER_W = _N // _NW
_CHUNK = 512
_NCHUNK = _PER_W // _CHUNK


def _make_gather():
    mesh = plsc.VectorSubcoreMesh(core_axis_name="c", subcore_axis_name="s")

    @functools.partial(
        pl.kernel,
        mesh=mesh,
        out_type=jax.ShapeDtypeStruct((_N, _DIM), jnp.float32),
        scratch_types=[
            pltpu.VMEM((_CHUNK,), jnp.int32),
            pltpu.VMEM((_CHUNK, _DIM), jnp.float32),
            pltpu.SemaphoreType.DMA,
        ],
        compiler_params=pltpu.CompilerParams(use_tc_tiling_on_sc=False),
    )
    def gather_kernel(table_hbm, idx_hbm, out_hbm, idx_v, rows_v, sem):
        wid = lax.axis_index("s") * 2 + lax.axis_index("c")
        base = wid * _PER_W

        def body(i, carry):
            off = base + i * _CHUNK
            pltpu.sync_copy(idx_hbm.at[pl.ds(off, _CHUNK)], idx_v)
            pltpu.async_copy(table_hbm.at[idx_v], rows_v, sem).wait()
            pltpu.sync_copy(rows_v, out_hbm.at[pl.ds(off, _CHUNK)])
            return carry

        lax.fori_loop(0, _NCHUNK, body, 0)

    return gather_kernel


_gather = _make_gather()


@jax.jit
def kernel(input, embd_weight):
    idx_flat = input.reshape(_N).astype(jnp.int32)
    out = _gather(embd_weight, idx_flat)
    return out.reshape(_B, _L, _DIM)


# speedup vs baseline: 6.2122x; 1.0708x over previous
import functools

import jax
import jax.numpy as jnp
from jax import lax
from jax.experimental import pallas as pl
from jax.experimental.pallas import tpu as pltpu
from jax.experimental.pallas import tpu_sc as plsc

_VOCAB = 100000
_DIM = 64
_B = 16384
_L = 50
_N = _B * _L
_NW = 32
_PER_W = _N // _NW
_CHUNK = 512
_NCHUNK = _PER_W // _CHUNK
_NBUF = 2
_NGROUP = _NCHUNK // _NBUF


def _make_gather():
    mesh = plsc.VectorSubcoreMesh(core_axis_name="c", subcore_axis_name="s")

    @functools.partial(
        pl.kernel,
        mesh=mesh,
        out_type=jax.ShapeDtypeStruct((_N, _DIM), jnp.float32),
        scratch_types=[
            pltpu.VMEM((_CHUNK,), jnp.int32),
            pltpu.VMEM((_CHUNK,), jnp.int32),
            pltpu.VMEM((_CHUNK, _DIM), jnp.float32),
            pltpu.VMEM((_CHUNK, _DIM), jnp.float32),
            pltpu.SemaphoreType.DMA,
            pltpu.SemaphoreType.DMA,
            pltpu.SemaphoreType.DMA,
            pltpu.SemaphoreType.DMA,
            pltpu.SemaphoreType.DMA,
            pltpu.SemaphoreType.DMA,
        ],
        compiler_params=pltpu.CompilerParams(use_tc_tiling_on_sc=False),
    )
    def gather_kernel(table_hbm, idx_hbm, out_hbm, idx_v0, idx_v1,
                      rows_v0, rows_v1,
                      isem0, isem1, gsem0, gsem1, osem0, osem1):
        idx_vs = (idx_v0, idx_v1)
        rows_vs = (rows_v0, rows_v1)
        isems = (isem0, isem1)
        gsems = (gsem0, gsem1)
        osems = (osem0, osem1)
        wid = lax.axis_index("s") * 2 + lax.axis_index("c")
        base = wid * _PER_W

        def idx_chunk(i):
            return idx_hbm.at[pl.ds(base + i * _CHUNK, _CHUNK)]

        def out_chunk(i):
            return out_hbm.at[pl.ds(base + i * _CHUNK, _CHUNK)]

        for b in range(_NBUF):
            pltpu.async_copy(idx_chunk(b), idx_vs[b], isems[b])

        def group(g, carry):
            for b in range(_NBUF):
                i = g * _NBUF + b
                pltpu.make_async_copy(idx_chunk(i), idx_vs[b], isems[b]).wait()

                @pl.when(g >= 1)
                def _():
                    pltpu.make_async_copy(rows_vs[b], out_chunk(i), osems[b]).wait()

                pltpu.async_copy(table_hbm.at[idx_vs[b]], rows_vs[b], gsems[b]).wait()

                @pl.when(g < _NGROUP - 1)
                def _():
                    pltpu.async_copy(idx_chunk(i + _NBUF), idx_vs[b], isems[b])

                pltpu.async_copy(rows_vs[b], out_chunk(i), osems[b])
            return carry

        lax.fori_loop(0, _NGROUP, group, 0)

        for b in range(_NBUF):
            pltpu.make_async_copy(rows_vs[b], out_chunk(b), osems[b]).wait()

    return gather_kernel


_gather = _make_gather()


@jax.jit
def kernel(input, embd_weight):
    idx_flat = input.reshape(_N).astype(jnp.int32)
    out = _gather(embd_weight, idx_flat)
    return out.reshape(_B, _L, _DIM)
